# baseline (speedup 1.0000x reference)
import math

import numpy as np
import jax
import jax.numpy as jnp
from jax.experimental import pallas as pl
from jax.experimental.pallas import tpu as pltpu

B, S, H, R, K, A = 4, 2048, 2048, 64, 8, 32
HD = 64
CH = 256
CPB = S // CH
NCHUNK = B * CPB
NBUF = 12


def _hadamard_np(n):
    if n == 1:
        return np.array([[1.0]], dtype=np.float64)
    h = _hadamard_np(n // 2)
    top = np.concatenate([h, h], axis=1)
    bot = np.concatenate([h, -h], axis=1)
    return np.concatenate([top, bot], axis=0) / math.sqrt(n)


_HMAT_T = np.ascontiguousarray(_hadamard_np(HD).T.astype(np.float32))


def _fused_kernel(hid_ref, act_ref, hmt_ref, w1_ref, b1_ref, lng_ref, lnb_ref,
                  w2_ref, b2_ref, ml_ref, rs_ref, out_ref,
                  buf_ref, actv_ref, outv_ref, insem, actsem, outsem):
    def start_in(c):
        pltpu.make_async_copy(
            hid_ref.at[c // CPB, pl.ds((c % CPB) * CH, CH), :],
            buf_ref.at[c % NBUF], insem.at[c % NBUF]).start()

    for c in range(NBUF):
        start_in(c)
    pltpu.make_async_copy(act_ref, actv_ref, actsem).start()

    iota_l = jax.lax.broadcasted_iota(jnp.int32, (R, R), 1)
    iota_s = jax.lax.broadcasted_iota(jnp.int32, (R, R), 0)

    for b in range(B):
        acc = None
        for j in range(CPB):
            c = b * CPB + j
            pltpu.make_async_copy(
                hid_ref.at[c // CPB, pl.ds((c % CPB) * CH, CH), :],
                buf_ref.at[c % NBUF], insem.at[c % NBUF]).wait()
            part = jnp.sum(buf_ref[c % NBUF].reshape(CH // 8, 8, H), axis=0)
            acc = part if acc is None else acc + part
            if c + NBUF < NCHUNK:
                start_in(c + NBUF)

        pooled = jnp.sum(acc, axis=0, keepdims=True) * (1.0 / S)
        h = jax.lax.dot_general(
            pooled, w1_ref[...], (((1,), (1,)), ((), ())),
            precision=jax.lax.Precision.HIGHEST,
            preferred_element_type=jnp.float32) + b1_ref[...]
        mu = jnp.mean(h, axis=-1, keepdims=True)
        var = jnp.mean((h - mu) ** 2, axis=-1, keepdims=True)
        h = (h - mu) * jax.lax.rsqrt(var + 1e-5) * lng_ref[...] + lnb_ref[...]
        h = h * 0.5 * (1.0 + jax.lax.erf(h * (1.0 / math.sqrt(2.0))))
        logits = jax.lax.dot_general(
            h, w2_ref[...], (((1,), (0,)), ((), ())),
            precision=jax.lax.Precision.HIGHEST,
            preferred_element_type=jnp.float32) + b2_ref[...]
        combined = logits + ml_ref[...]

        col = jnp.broadcast_to(combined, (R, R))
        row = jnp.transpose(combined).reshape(R, 1)
        beats = (row > col) | ((row == col) & (iota_s < iota_l))
        rank = jnp.sum(beats.astype(jnp.float32), axis=0, keepdims=True)
        mask = (rank < K).astype(jnp.float32)

        w = mask * rs_ref[...]

        mw = hmt_ref[...] * w
        if b == 0:
            pltpu.make_async_copy(act_ref, actv_ref, actsem).wait()
        outv_ref[b] = jax.lax.dot_general(
            mw, actv_ref[b], (((1,), (0,)), ((), ())),
            preferred_element_type=jnp.float32)
        pltpu.make_async_copy(outv_ref.at[b], out_ref.at[b], outsem.at[b]).start()

    for b in range(B):
        pltpu.make_async_copy(outv_ref.at[b], out_ref.at[b], outsem.at[b]).wait()


def kernel(rank_activations, hidden_states, W1, b1, ln_g, ln_b, W2, b2, mask_logits, rank_scales):
    hmt = jnp.asarray(_HMAT_T)
    out = pl.pallas_call(
        _fused_kernel,
        in_specs=[
            pl.BlockSpec(memory_space=pl.ANY),
            pl.BlockSpec(memory_space=pl.ANY),
            pl.BlockSpec((HD, HD), lambda: (0, 0)),
            pl.BlockSpec((A, H), lambda: (0, 0)),
            pl.BlockSpec((1, A), lambda: (0, 0)),
            pl.BlockSpec((1, A), lambda: (0, 0)),
            pl.BlockSpec((1, A), lambda: (0, 0)),
            pl.BlockSpec((A, R), lambda: (0, 0)),
            pl.BlockSpec((1, R), lambda: (0, 0)),
            pl.BlockSpec((1, R), lambda: (0, 0)),
            pl.BlockSpec((1, R), lambda: (0, 0)),
        ],
        out_specs=pl.BlockSpec(memory_space=pl.ANY),
        out_shape=jax.ShapeDtypeStruct((B, R, S), jnp.float32),
        scratch_shapes=[
            pltpu.VMEM((NBUF, CH, H), jnp.float32),
            pltpu.VMEM((B, R, S), jnp.float32),
            pltpu.VMEM((B, R, S), jnp.float32),
            pltpu.SemaphoreType.DMA((NBUF,)),
            pltpu.SemaphoreType.DMA(()),
            pltpu.SemaphoreType.DMA((B,)),
        ],
    )(
        hidden_states, rank_activations.transpose(0, 2, 1), hmt, W1,
        b1.reshape(1, A), ln_g.reshape(1, A), ln_b.reshape(1, A),
        jnp.transpose(W2), b2.reshape(1, R), mask_logits.reshape(1, R),
        rank_scales.reshape(1, R),
    )
    return out.transpose(0, 2, 1)


# speedup vs baseline: 1.0286x; 1.0286x over previous
import math

import numpy as np
import jax
import jax.numpy as jnp
from jax.experimental import pallas as pl
from jax.experimental.pallas import tpu as pltpu

B, S, H, R, K, A = 4, 2048, 2048, 64, 8, 32
HD = 64
CH = 256
CPB = S // CH
NCHUNK = B * CPB
NBUF = 12


def _hadamard_np(n):
    if n == 1:
        return np.array([[1.0]], dtype=np.float64)
    h = _hadamard_np(n // 2)
    top = np.concatenate([h, h], axis=1)
    bot = np.concatenate([h, -h], axis=1)
    return np.concatenate([top, bot], axis=0) / math.sqrt(n)


_HMAT_T = np.ascontiguousarray(_hadamard_np(HD).T.astype(np.float32))


def _fused_kernel(hid_ref, act_ref, hmt_ref, w1_ref, b1_ref, lng_ref, lnb_ref,
                  w2_ref, b2_ref, ml_ref, rs_ref, out_ref,
                  buf_ref, actv_ref, outv_ref, insem, actsem, outsem):
    def start_in(c):
        for hh in range(2):
            pltpu.make_async_copy(
                hid_ref.at[c // CPB, pl.ds((c % CPB) * CH, CH),
                           pl.ds(hh * (H // 2), H // 2)],
                buf_ref.at[c % NBUF, :, pl.ds(hh * (H // 2), H // 2)],
                insem.at[c % NBUF, hh]).start()

    def wait_in(c):
        for hh in range(2):
            pltpu.make_async_copy(
                hid_ref.at[c // CPB, pl.ds((c % CPB) * CH, CH),
                           pl.ds(hh * (H // 2), H // 2)],
                buf_ref.at[c % NBUF, :, pl.ds(hh * (H // 2), H // 2)],
                insem.at[c % NBUF, hh]).wait()

    for c in range(NBUF):
        start_in(c)
    pltpu.make_async_copy(act_ref, actv_ref, actsem).start()

    iota_l = jax.lax.broadcasted_iota(jnp.int32, (R, R), 1)
    iota_s = jax.lax.broadcasted_iota(jnp.int32, (R, R), 0)

    for b in range(B):
        acc = None
        for j in range(CPB):
            c = b * CPB + j
            wait_in(c)
            part = jnp.sum(buf_ref[c % NBUF].reshape(CH // 8, 8, H), axis=0)
            acc = part if acc is None else acc + part
            if c + NBUF < NCHUNK:
                start_in(c + NBUF)

        pooled = jnp.sum(acc, axis=0, keepdims=True) * (1.0 / S)
        h = jax.lax.dot_general(
            pooled, w1_ref[...], (((1,), (1,)), ((), ())),
            precision=jax.lax.Precision.HIGHEST,
            preferred_element_type=jnp.float32) + b1_ref[...]
        mu = jnp.mean(h, axis=-1, keepdims=True)
        var = jnp.mean((h - mu) ** 2, axis=-1, keepdims=True)
        h = (h - mu) * jax.lax.rsqrt(var + 1e-5) * lng_ref[...] + lnb_ref[...]
        h = h * 0.5 * (1.0 + jax.lax.erf(h * (1.0 / math.sqrt(2.0))))
        logits = jax.lax.dot_general(
            h, w2_ref[...], (((1,), (0,)), ((), ())),
            precision=jax.lax.Precision.HIGHEST,
            preferred_element_type=jnp.float32) + b2_ref[...]
        combined = logits + ml_ref[...]

        col = jnp.broadcast_to(combined, (R, R))
        row = jnp.transpose(combined).reshape(R, 1)
        beats = (row > col) | ((row == col) & (iota_s < iota_l))
        rank = jnp.sum(beats.astype(jnp.float32), axis=0, keepdims=True)
        mask = (rank < K).astype(jnp.float32)

        w = mask * rs_ref[...]

        mw = hmt_ref[...] * w
        if b == 0:
            pltpu.make_async_copy(act_ref, actv_ref, actsem).wait()
        outv_ref[b] = jax.lax.dot_general(
            mw, actv_ref[b], (((1,), (0,)), ((), ())),
            preferred_element_type=jnp.float32)
        pltpu.make_async_copy(outv_ref.at[b], out_ref.at[b], outsem.at[b]).start()

    for b in range(B):
        pltpu.make_async_copy(outv_ref.at[b], out_ref.at[b], outsem.at[b]).wait()


def kernel(rank_activations, hidden_states, W1, b1, ln_g, ln_b, W2, b2, mask_logits, rank_scales):
    hmt = jnp.asarray(_HMAT_T)
    out = pl.pallas_call(
        _fused_kernel,
        in_specs=[
            pl.BlockSpec(memory_space=pl.ANY),
            pl.BlockSpec(memory_space=pl.ANY),
            pl.BlockSpec((HD, HD), lambda: (0, 0)),
            pl.BlockSpec((A, H), lambda: (0, 0)),
            pl.BlockSpec((1, A), lambda: (0, 0)),
            pl.BlockSpec((1, A), lambda: (0, 0)),
            pl.BlockSpec((1, A), lambda: (0, 0)),
            pl.BlockSpec((A, R), lambda: (0, 0)),
            pl.BlockSpec((1, R), lambda: (0, 0)),
            pl.BlockSpec((1, R), lambda: (0, 0)),
            pl.BlockSpec((1, R), lambda: (0, 0)),
        ],
        out_specs=pl.BlockSpec(memory_space=pl.ANY),
        out_shape=jax.ShapeDtypeStruct((B, R, S), jnp.float32),
        scratch_shapes=[
            pltpu.VMEM((NBUF, CH, H), jnp.float32),
            pltpu.VMEM((B, R, S), jnp.float32),
            pltpu.VMEM((B, R, S), jnp.float32),
            pltpu.SemaphoreType.DMA((NBUF, 2)),
            pltpu.SemaphoreType.DMA(()),
            pltpu.SemaphoreType.DMA((B,)),
        ],
    )(
        hidden_states, rank_activations.transpose(0, 2, 1), hmt, W1,
        b1.reshape(1, A), ln_g.reshape(1, A), ln_b.reshape(1, A),
        jnp.transpose(W2), b2.reshape(1, R), mask_logits.reshape(1, R),
        rank_scales.reshape(1, R),
    )
    return out.transpose(0, 2, 1)
